# initial kernel scaffold (unmeasured)
import jax
import jax.numpy as jnp
from jax import lax
from jax.experimental import pallas as pl
from jax.experimental.pallas import tpu as pltpu

N_DEV = 4
SQ = 1024
D_MODEL = 1024
HQ_LOC = 8
DH = 128
SCALE = 0.08838834764831843
NEG = -1e9


def kernel(x, Wq, K_ext, V_ext, Wo):
    i = lax.axis_index("i")
    x2 = x.reshape(SQ, D_MODEL)
    K_loc = lax.dynamic_slice_in_dim(K_ext, i * HQ_LOC, HQ_LOC, axis=2)
    V_loc = lax.dynamic_slice_in_dim(V_ext, i * HQ_LOC, HQ_LOC, axis=2)
    K_t = jnp.transpose(K_loc, (0, 2, 1, 3))
    V_t = jnp.transpose(V_loc, (0, 2, 1, 3))

    def body(x_ref, wq_ref, k_ref, v_ref, wo_ref, out_ref,
             xg_ref, part_ref, rs_ref, ag_send, ag_recv, rs_send, rs_recv):
        my = lax.axis_index("i")
        left = lax.rem(my + (N_DEV - 1), N_DEV)
        right = lax.rem(my + 1, N_DEV)

        barrier_sem = pltpu.get_barrier_semaphore()
        for nbr in (left, right):
            pl.semaphore_signal(barrier_sem, inc=1, device_id=(nbr,),
                                device_id_type=pl.DeviceIdType.MESH)
        pl.semaphore_wait(barrier_sem, 2)

        for h in range(N_DEV - 1):
            src = x_ref if h == 0 else xg_ref.at[h - 1]
            rdma = pltpu.make_async_remote_copy(
                src_ref=src,
                dst_ref=xg_ref.at[h],
                send_sem=ag_send.at[h],
                recv_sem=ag_recv.at[h],
                device_id=(right,),
                device_id_type=pl.DeviceIdType.MESH,
            )
            rdma.start()
            rdma.wait()

        qi = lax.broadcasted_iota(jnp.int32, (SQ, SQ), 0)
        ki = lax.broadcasted_iota(jnp.int32, (SQ, SQ), 1)
        mask = (jnp.abs(qi - ki) <= 128) | (ki < 32) | (qi < 32)

        def compute_partial(x2d, b, slot):
            q = jnp.dot(x2d, wq_ref[...], preferred_element_type=jnp.float32)
            acc = jnp.zeros((SQ, D_MODEL), jnp.float32)
            for hh in range(HQ_LOC):
                qh = q[:, hh * DH:(hh + 1) * DH]
                kh = k_ref[b, hh]
                s = lax.dot_general(qh, kh, (((1,), (1,)), ((), ())),
                                    preferred_element_type=jnp.float32)
                s = jnp.where(mask, s * SCALE, NEG)
                m = jnp.max(s, axis=1, keepdims=True)
                w = jnp.exp(s - m)
                w = w / jnp.sum(w, axis=1, keepdims=True)
                ctx = jnp.dot(w, v_ref[b, hh],
                              preferred_element_type=jnp.float32)
                acc = acc + jnp.dot(ctx, wo_ref[hh * DH:(hh + 1) * DH, :],
                                    preferred_element_type=jnp.float32)
            part_ref[slot] = acc

        compute_partial(x_ref[...], my, 0)
        for h in range(1, N_DEV):
            b = lax.rem(my - h + N_DEV, N_DEV)
            compute_partial(xg_ref[h - 1], b, h)

        for s in range(N_DEV - 1):
            rdma = pltpu.make_async_remote_copy(
                src_ref=part_ref.at[s + 1],
                dst_ref=rs_ref.at[s],
                send_sem=rs_send.at[s],
                recv_sem=rs_recv.at[s],
                device_id=(right,),
                device_id_type=pl.DeviceIdType.MESH,
            )
            rdma.start()
            rdma.wait()
            tgt = (s + 2) % N_DEV
            part_ref[tgt] = part_ref[tgt] + rs_ref[s]

        out_ref[...] = part_ref[0]

    out = pl.pallas_call(
        body,
        out_shape=jax.ShapeDtypeStruct((SQ, D_MODEL), jnp.float32),
        in_specs=[pl.BlockSpec(memory_space=pltpu.VMEM)] * 5,
        out_specs=pl.BlockSpec(memory_space=pltpu.VMEM),
        scratch_shapes=[
            pltpu.VMEM((N_DEV - 1, SQ, D_MODEL), jnp.float32),
            pltpu.VMEM((N_DEV, SQ, D_MODEL), jnp.float32),
            pltpu.VMEM((N_DEV - 1, SQ, D_MODEL), jnp.float32),
            pltpu.SemaphoreType.DMA((N_DEV - 1,)),
            pltpu.SemaphoreType.DMA((N_DEV - 1,)),
            pltpu.SemaphoreType.DMA((N_DEV - 1,)),
            pltpu.SemaphoreType.DMA((N_DEV - 1,)),
        ],
        compiler_params=pltpu.CompilerParams(collective_id=0),
    )(x2, Wq, K_t, V_t, Wo)
    return out.reshape(1, SQ, D_MODEL)


# baseline (device time: 494468 ns/iter reference)
import jax
import jax.numpy as jnp
from jax import lax
from jax.experimental import pallas as pl
from jax.experimental.pallas import tpu as pltpu

N_DEV = 4
SQ = 1024
D_MODEL = 1024
HQ_LOC = 8
DH = 128
SCALE = 0.08838834764831843
NEG = -1e9


def kernel(x, Wq, K_ext, V_ext, Wo):
    i = lax.axis_index("i")
    x2 = x.reshape(SQ, D_MODEL)
    K_my = jnp.transpose(K_ext[i], (1, 0, 2))
    V_my = jnp.transpose(V_ext[i], (1, 0, 2))
    W_pair = jnp.stack([Wq, Wo])

    def body(x_ref, w_ref, k_hbm, v_hbm, out_ref,
             wbuf, kscr, vscr, bias_ref,
             send_sems, recv_sems, kv_sems, credit_sem):
        my = lax.axis_index("i")
        left = lax.rem(my + (N_DEV - 1), N_DEV)
        right = lax.rem(my + 1, N_DEV)

        barrier_sem = pltpu.get_barrier_semaphore()
        for nbr in (left, right):
            pl.semaphore_signal(barrier_sem, inc=1, device_id=(nbr,),
                                device_id_type=pl.DeviceIdType.MESH)
        pl.semaphore_wait(barrier_sem, 2)

        qi = lax.broadcasted_iota(jnp.int32, (SQ, SQ), 0)
        ki = lax.broadcasted_iota(jnp.int32, (SQ, SQ), 1)
        mask = (jnp.abs(qi - ki) <= 128) | (ki < 32) | (qi < 32)
        bias_ref[...] = jnp.where(mask, 0.0, NEG)

        def fetch_kv(group):
            kc = pltpu.make_async_copy(
                k_hbm.at[pl.ds(group * HQ_LOC, HQ_LOC)], kscr, kv_sems.at[0])
            vc = pltpu.make_async_copy(
                v_hbm.at[pl.ds(group * HQ_LOC, HQ_LOC)], vscr, kv_sems.at[1])
            kc.start()
            vc.start()
            return kc, vc

        def compute_group(w_src):
            def head_body(hh, carry):
                c = pl.ds(hh * DH, DH)
                qh = jnp.dot(x_ref[...], w_src[0, :, c],
                             preferred_element_type=jnp.float32)
                s = lax.dot_general(qh, kscr[hh], (((1,), (1,)), ((), ())),
                                    preferred_element_type=jnp.float32)
                s = s * SCALE + bias_ref[...]
                m = jnp.max(s, axis=1, keepdims=True)
                w = jnp.exp(s - m)
                d = jnp.sum(w, axis=1, keepdims=True)
                ctx = jnp.dot(w, vscr[hh],
                              preferred_element_type=jnp.float32) / d
                out_ref[...] = out_ref[...] + jnp.dot(
                    ctx, w_src[1, c, :],
                    preferred_element_type=jnp.float32)
                return carry

            lax.fori_loop(0, HQ_LOC, head_body, 0)

        kc, vc = fetch_kv(my)
        kc.wait()
        vc.wait()
        out_ref[...] = jnp.zeros((SQ, D_MODEL), jnp.float32)
        compute_group(w_ref)

        for h in range(N_DEV - 1):
            src = w_ref if h == 0 else wbuf.at[(h - 1) % 2]
            if h == 2:
                pl.semaphore_wait(credit_sem, 1)
            rdma = pltpu.make_async_remote_copy(
                src_ref=src,
                dst_ref=wbuf.at[h % 2],
                send_sem=send_sems.at[h],
                recv_sem=recv_sems.at[h],
                device_id=(right,),
                device_id_type=pl.DeviceIdType.MESH,
            )
            rdma.start()
            g = lax.rem(my - h - 1 + N_DEV, N_DEV)
            kc, vc = fetch_kv(g)
            rdma.wait()
            kc.wait()
            vc.wait()
            compute_group(wbuf.at[h % 2])
            if h == 1:
                pl.semaphore_signal(credit_sem, inc=1, device_id=(left,),
                                    device_id_type=pl.DeviceIdType.MESH)

    out = pl.pallas_call(
        body,
        out_shape=jax.ShapeDtypeStruct((SQ, D_MODEL), jnp.float32),
        in_specs=[
            pl.BlockSpec(memory_space=pltpu.VMEM),
            pl.BlockSpec(memory_space=pltpu.VMEM),
            pl.BlockSpec(memory_space=pltpu.MemorySpace.HBM),
            pl.BlockSpec(memory_space=pltpu.MemorySpace.HBM),
        ],
        out_specs=pl.BlockSpec(memory_space=pltpu.VMEM),
        scratch_shapes=[
            pltpu.VMEM((2, 2, D_MODEL, D_MODEL), jnp.float32),
            pltpu.VMEM((HQ_LOC, SQ, DH), jnp.float32),
            pltpu.VMEM((HQ_LOC, SQ, DH), jnp.float32),
            pltpu.VMEM((SQ, SQ), jnp.float32),
            pltpu.SemaphoreType.DMA((N_DEV - 1,)),
            pltpu.SemaphoreType.DMA((N_DEV - 1,)),
            pltpu.SemaphoreType.DMA((2,)),
            pltpu.SemaphoreType.REGULAR,
        ],
        compiler_params=pltpu.CompilerParams(
            collective_id=0,
            vmem_limit_bytes=47 * 1024 * 1024,
        ),
    )(x2, W_pair, K_my, V_my)
    return out.reshape(1, SQ, D_MODEL)


# device time: 222598 ns/iter; 2.2213x vs baseline; 2.2213x over previous
import jax
import jax.numpy as jnp
from jax import lax
from jax.experimental import pallas as pl
from jax.experimental.pallas import tpu as pltpu

N_DEV = 4
SQ = 1024
D_MODEL = 1024
HQ_LOC = 8
DH = 128
SCALE = 0.08838834764831843
NEG = -1e9
BF = jnp.bfloat16


def kernel(x, Wq, K_ext, V_ext, Wo):
    i = lax.axis_index("i")
    x2 = x.reshape(SQ, D_MODEL).astype(BF)
    K_my = jnp.transpose(K_ext[i], (1, 0, 2)).astype(BF)
    V_my = jnp.transpose(V_ext[i], (1, 0, 2)).astype(BF)
    W_pair = jnp.stack([Wq, Wo]).astype(BF)

    def body(x_ref, w_ref, k_hbm, v_hbm, out_ref,
             wbuf, kscr, vscr, bias_ref,
             send_sems, recv_sems, kv_sems, credit_sem):
        my = lax.axis_index("i")
        left = lax.rem(my + (N_DEV - 1), N_DEV)
        right = lax.rem(my + 1, N_DEV)

        def fetch_kv(group, slot):
            kc = pltpu.make_async_copy(
                k_hbm.at[pl.ds(group * HQ_LOC, HQ_LOC)], kscr.at[slot],
                kv_sems.at[slot, 0])
            vc = pltpu.make_async_copy(
                v_hbm.at[pl.ds(group * HQ_LOC, HQ_LOC)], vscr.at[slot],
                kv_sems.at[slot, 1])
            kc.start()
            vc.start()
            return kc, vc

        kv = fetch_kv(my, 0)

        barrier_sem = pltpu.get_barrier_semaphore()
        for nbr in (left, right):
            pl.semaphore_signal(barrier_sem, inc=1, device_id=(nbr,),
                                device_id_type=pl.DeviceIdType.MESH)
        pl.semaphore_wait(barrier_sem, 2)

        qi = lax.broadcasted_iota(jnp.int32, (SQ, SQ), 0)
        ki = lax.broadcasted_iota(jnp.int32, (SQ, SQ), 1)
        mask = (jnp.abs(qi - ki) <= 128) | (ki < 32) | (qi < 32)
        bias_ref[...] = jnp.where(mask, 0.0, NEG)

        def compute_group(w_src, slot):
            def head_body(hh, carry):
                c = pl.ds(hh * DH, DH)
                qh = jnp.dot(x_ref[...], w_src[0, :, c],
                             preferred_element_type=jnp.float32)
                s = lax.dot_general(qh.astype(BF), kscr[slot, hh],
                                    (((1,), (1,)), ((), ())),
                                    preferred_element_type=jnp.float32)
                s = s * SCALE + bias_ref[...]
                m = jnp.max(s, axis=1, keepdims=True)
                w = jnp.exp(s - m)
                d = jnp.sum(w, axis=1, keepdims=True)
                ctx = jnp.dot(w.astype(BF), vscr[slot, hh],
                              preferred_element_type=jnp.float32) / d
                out_ref[...] = out_ref[...] + jnp.dot(
                    ctx.astype(BF), w_src[1, c, :],
                    preferred_element_type=jnp.float32)
                return carry

            lax.fori_loop(0, HQ_LOC, head_body, 0)

        def rdma_hop(h, src):
            r = pltpu.make_async_remote_copy(
                src_ref=src,
                dst_ref=wbuf.at[h % 2],
                send_sem=send_sems.at[h],
                recv_sem=recv_sems.at[h],
                device_id=(right,),
                device_id_type=pl.DeviceIdType.MESH,
            )
            r.start()
            return r

        r0 = rdma_hop(0, w_ref)
        kv[0].wait()
        kv[1].wait()
        kv_next = fetch_kv(lax.rem(my + 3, N_DEV), 1)
        out_ref[...] = jnp.zeros((SQ, D_MODEL), jnp.float32)
        compute_group(w_ref, 0)
        r0.wait()

        r1 = rdma_hop(1, wbuf.at[0])
        kv_next[0].wait()
        kv_next[1].wait()
        kv_next = fetch_kv(lax.rem(my + 2, N_DEV), 0)
        compute_group(wbuf.at[0], 1)
        r1.wait()
        pl.semaphore_signal(credit_sem, inc=1, device_id=(left,),
                            device_id_type=pl.DeviceIdType.MESH)

        pl.semaphore_wait(credit_sem, 1)
        r2 = rdma_hop(2, wbuf.at[1])
        kv_next[0].wait()
        kv_next[1].wait()
        kv_next = fetch_kv(lax.rem(my + 1, N_DEV), 1)
        compute_group(wbuf.at[1], 0)
        r2.wait()

        kv_next[0].wait()
        kv_next[1].wait()
        compute_group(wbuf.at[0], 1)

    out = pl.pallas_call(
        body,
        out_shape=jax.ShapeDtypeStruct((SQ, D_MODEL), jnp.float32),
        in_specs=[
            pl.BlockSpec(memory_space=pltpu.VMEM),
            pl.BlockSpec(memory_space=pltpu.VMEM),
            pl.BlockSpec(memory_space=pltpu.MemorySpace.HBM),
            pl.BlockSpec(memory_space=pltpu.MemorySpace.HBM),
        ],
        out_specs=pl.BlockSpec(memory_space=pltpu.VMEM),
        scratch_shapes=[
            pltpu.VMEM((2, 2, D_MODEL, D_MODEL), BF),
            pltpu.VMEM((2, HQ_LOC, SQ, DH), BF),
            pltpu.VMEM((2, HQ_LOC, SQ, DH), BF),
            pltpu.VMEM((SQ, SQ), jnp.float32),
            pltpu.SemaphoreType.DMA((N_DEV - 1,)),
            pltpu.SemaphoreType.DMA((N_DEV - 1,)),
            pltpu.SemaphoreType.DMA((2, 2)),
            pltpu.SemaphoreType.REGULAR,
        ],
        compiler_params=pltpu.CompilerParams(
            collective_id=0,
            vmem_limit_bytes=44 * 1024 * 1024,
        ),
    )(x2, W_pair, K_my, V_my)
    return out.reshape(1, SQ, D_MODEL)
